# Initial kernel scaffold; baseline (speedup 1.0000x reference)
import jax
import jax.numpy as jnp
from jax.experimental import pallas as pl


def kernel(x, edge_index, params):
    raise NotImplementedError("write your pallas kernel here")


# speedup vs baseline: 4.0838x; 4.0838x over previous
import functools

import jax
import jax.numpy as jnp
from jax import lax
from jax.experimental import pallas as pl
from jax.experimental.pallas import tpu as pltpu
from jax.experimental.pallas import tpu_sc as plsc

N_NODES = 10000
N_EDGES = 160000
IN_CH = 256
HID = 512
OUT_CH = 256
CHUNK = 64
NUM_LAYERS = 8
REP = HID // CHUNK

NPAD = 10240
RB = 256
NBLK = NPAD // RB
NQ = 4
QF = HID // NQ
QPC = NQ // 2

NC = 2
NS = 16
EPT = N_EDGES // NS
EK = 200
NCH = EPT // EK
ACC_ROWS = 10112
ACC_PT = ACC_ROWS // NS
DEG_PT = NPAD // NS


def _ln_blk(h, g, b, eps=1e-5):
    mu = jnp.mean(h, axis=-1, keepdims=True)
    d = h - mu
    var = jnp.mean(d * d, axis=-1, keepdims=True)
    return d / jnp.sqrt(var + eps) * g + b


def _dot(a, b):
    return jnp.dot(a, b, preferred_element_type=jnp.float32)


def _full(shape):
    nd = len(shape)
    return pl.BlockSpec(shape, lambda i: (0,) * nd)


def _planes(ref):
    return jnp.concatenate([ref[q] for q in range(NQ)], axis=-1)


def _store_planes(ref, val):
    for q in range(NQ):
        ref[q] = val[:, q * QF:(q + 1) * QF]


_PLANE_SPEC = pl.BlockSpec((NQ, RB, QF), lambda i: (0, i, 0))


def _stem_body(x_ref, w1, b1, g1, gb1, w2, b2, g2, gb2, o_ref):
    h = x_ref[...]
    h = jax.nn.gelu(_dot(h, w1[...]) + b1[...])
    h = _ln_blk(h, g1[...], gb1[...])
    h = jax.nn.gelu(_dot(h, w2[...]) + b2[...])
    h = _ln_blk(h, g2[...], gb2[...])
    _store_planes(o_ref, h + h)


def _stem(x, p):
    row = lambda a: a.reshape(1, -1)
    return pl.pallas_call(
        _stem_body,
        grid=(NBLK,),
        in_specs=[
            pl.BlockSpec((RB, IN_CH), lambda i: (i, 0)),
            _full((IN_CH, HID)), _full((1, HID)), _full((1, HID)), _full((1, HID)),
            _full((HID, HID)), _full((1, HID)), _full((1, HID)), _full((1, HID)),
        ],
        out_specs=_PLANE_SPEC,
        out_shape=jax.ShapeDtypeStruct((NQ, NPAD, QF), jnp.float32),
    )(x,
      p["lin_in"][0]["W"], row(p["lin_in"][0]["b"]),
      row(p["norm_in"][0]["g"]), row(p["norm_in"][0]["b"]),
      p["lin_in"][1]["W"], row(p["lin_in"][1]["b"]),
      row(p["norm_in"][1]["g"]), row(p["norm_in"][1]["b"]))


def _out_body(x_ref, w1, b1, w2, b2, o_ref):
    h = jax.nn.gelu(_dot(_planes(x_ref), w1[...]) + b1[...])
    o_ref[...] = _dot(h, w2[...]) + b2[...]


def _out_mlp(xq, p):
    row = lambda a: a.reshape(1, -1)
    return pl.pallas_call(
        _out_body,
        grid=(NBLK,),
        in_specs=[
            _PLANE_SPEC,
            _full((HID, HID)), _full((1, HID)),
            _full((HID, OUT_CH)), _full((1, OUT_CH)),
        ],
        out_specs=pl.BlockSpec((RB, OUT_CH), lambda i: (i, 0)),
        out_shape=jax.ShapeDtypeStruct((NPAD, OUT_CH), jnp.float32),
    )(xq, p["lin_out"][0]["W"], row(p["lin_out"][0]["b"]),
      p["lin_out"][1]["W"], row(p["lin_out"][1]["b"]))


def _layer_body(x_ref, y_ref, m4_ref, deg_ref, fr_ref,
                wi1, bi1, wi2, bi2, wf1, bf1, wf2, bf2,
                wo1, bo1, wo2, bo2, g, b, rep_ref,
                xo_ref, fro_ref):
    x = _planes(x_ref)
    msum = _planes(m4_ref)
    deg = deg_ref[...][:, 0:1]
    m = msum * (1.0 / jnp.clip(deg, 1.0, None))
    xm = jnp.concatenate([x, m], axis=-1)

    h = jax.nn.leaky_relu(_dot(xm, wi1[...]) + bi1[...], 0.01)
    in_sig = jax.nn.sigmoid(_dot(h, wi2[...]) + bi2[...])
    h = jax.nn.leaky_relu(_dot(xm, wf1[...]) + bf1[...], 0.01)
    fr_sig = jax.nn.sigmoid(_dot(h, wf2[...]) + bf2[...])

    fr_old = fr_ref[...]
    fr_new = fr_old + (1.0 - fr_old) * fr_sig
    rep = rep_ref[...]
    fr512 = _dot(fr_new, rep)
    ing512 = _dot(in_sig, rep)

    h = jax.nn.leaky_relu(_dot(m, wo1[...]) + bo1[...], 0.01)
    op = _dot(h, wo2[...]) + bo2[...]

    out = x * fr512 + op * (1.0 - fr512) * ing512 + _planes(y_ref)
    _store_planes(xo_ref, _ln_blk(out, g[...], b[...]))
    fro_ref[...] = fr_new


def _layer(xq, yq, m4, deg8, fr, conv, rep_mat):
    row = lambda a: a.reshape(1, -1)
    return pl.pallas_call(
        _layer_body,
        grid=(NBLK,),
        in_specs=[
            _PLANE_SPEC,
            _PLANE_SPEC,
            _PLANE_SPEC,
            pl.BlockSpec((RB, QF), lambda i: (i, 0)),
            pl.BlockSpec((RB, CHUNK), lambda i: (i, 0)),
            _full((2 * HID, CHUNK)), _full((1, CHUNK)),
            _full((CHUNK, CHUNK)), _full((1, CHUNK)),
            _full((2 * HID, CHUNK)), _full((1, CHUNK)),
            _full((CHUNK, CHUNK)), _full((1, CHUNK)),
            _full((HID, HID)), _full((1, HID)),
            _full((HID, HID)), _full((1, HID)),
            _full((1, HID)), _full((1, HID)),
            _full((CHUNK, HID)),
        ],
        out_specs=[
            _PLANE_SPEC,
            pl.BlockSpec((RB, CHUNK), lambda i: (i, 0)),
        ],
        out_shape=[
            jax.ShapeDtypeStruct((NQ, NPAD, QF), jnp.float32),
            jax.ShapeDtypeStruct((NPAD, CHUNK), jnp.float32),
        ],
    )(xq, yq, m4, deg8, fr,
      conv["in_net"][0]["W"], row(conv["in_net"][0]["b"]),
      conv["in_net"][1]["W"], row(conv["in_net"][1]["b"]),
      conv["fr_net"][0]["W"], row(conv["fr_net"][0]["b"]),
      conv["fr_net"][1]["W"], row(conv["fr_net"][1]["b"]),
      conv["op_net"][0]["W"], row(conv["op_net"][0]["b"]),
      conv["op_net"][1]["W"], row(conv["op_net"][1]["b"]),
      row(conv["tm_norm"]["g"]), row(conv["tm_norm"]["b"]),
      rep_mat)


def _deg_kernel(dst):
    mesh = plsc.VectorSubcoreMesh(core_axis_name="c", subcore_axis_name="s")

    @functools.partial(
        pl.kernel,
        out_type=jax.ShapeDtypeStruct((NPAD, QF), jnp.float32),
        mesh=mesh,
        scratch_types=[
            pltpu.VMEM((EK,), jnp.int32),
            pltpu.VMEM((EK, QF), jnp.float32),
            pltpu.VMEM_SHARED((NPAD, QF), jnp.float32),
        ],
    )
    def run(dst_hbm, zeros_hbm, ones_hbm, out_hbm, dst_v, ones_v, acc):
        c = lax.axis_index("c")
        s = lax.axis_index("s")

        @pl.when(c == 0)
        def _():
            pltpu.sync_copy(zeros_hbm, acc.at[pl.ds(s * DEG_PT, DEG_PT)])
            plsc.subcore_barrier()
            pltpu.sync_copy(ones_hbm, ones_v)

            @pl.loop(0, NCH)
            def _(ch):
                base = s * EPT + ch * EK
                pltpu.sync_copy(dst_hbm.at[pl.ds(base, EK)], dst_v)
                pltpu.sync_copy(ones_v, acc.at[dst_v], add=True)

            plsc.subcore_barrier()
            pltpu.sync_copy(acc.at[pl.ds(s * DEG_PT, DEG_PT)],
                            out_hbm.at[pl.ds(s * DEG_PT, DEG_PT)])

    zeros = jnp.zeros((DEG_PT, QF), jnp.float32)
    ones = jnp.ones((EK, QF), jnp.float32)
    return run(dst, zeros, ones)


def _segsum(xflat, idxq, dst):
    mesh = plsc.VectorSubcoreMesh(core_axis_name="c", subcore_axis_name="s")

    @functools.partial(
        pl.kernel,
        out_type=jax.ShapeDtypeStruct((NQ * NPAD, QF), jnp.float32),
        mesh=mesh,
        scratch_types=[
            pltpu.VMEM((EK,), jnp.int32),
            pltpu.VMEM((EK,), jnp.int32),
            pltpu.VMEM((EK, QF), jnp.float32),
            pltpu.VMEM_SHARED((ACC_ROWS, QF), jnp.float32),
            pltpu.SemaphoreType.DMA,
        ],
    )
    def run(x_hbm, idxq_hbm, dst_hbm, zeros_hbm, out_hbm,
            idx_v, dst_v, rows_v, acc, sem):
        c = lax.axis_index("c")
        s = lax.axis_index("s")
        for qi in range(QPC):
            q = c * QPC + qi
            pltpu.sync_copy(zeros_hbm, acc.at[pl.ds(s * ACC_PT, ACC_PT)])
            plsc.subcore_barrier()

            @pl.loop(0, NCH)
            def _(ch):
                base = s * EPT + ch * EK
                pltpu.sync_copy(idxq_hbm.at[pl.ds(q * N_EDGES + base, EK)],
                                idx_v)
                pltpu.sync_copy(dst_hbm.at[pl.ds(base, EK)], dst_v)
                pltpu.async_copy(x_hbm.at[idx_v], rows_v, sem).wait()
                pltpu.sync_copy(rows_v, acc.at[dst_v], add=True)

            plsc.subcore_barrier()
            pltpu.sync_copy(acc.at[pl.ds(s * ACC_PT, ACC_PT)],
                            out_hbm.at[pl.ds(q * NPAD + s * ACC_PT, ACC_PT)])
            plsc.subcore_barrier()

    zeros = jnp.zeros((ACC_PT, QF), jnp.float32)
    return run(xflat, idxq, dst, zeros)


def kernel(x, edge_index, params):
    src = edge_index[0]
    dst = edge_index[1]
    idxq = (jnp.arange(NQ, dtype=jnp.int32)[:, None] * NPAD + src[None, :]
            ).reshape(-1)

    deg8 = _deg_kernel(dst)

    xp = jnp.pad(x, ((0, NPAD - N_NODES), (0, 0)))
    hq = _stem(xp, params)
    yq = hq

    rep_mat = jnp.repeat(jnp.eye(CHUNK, dtype=jnp.float32), REP, axis=1)
    fr = jnp.zeros((NPAD, CHUNK), jnp.float32)
    signals = []
    for conv in params["convs"]:
        msum = _segsum(hq.reshape(NQ * NPAD, QF), idxq, dst)
        m4 = msum.reshape(NQ, NPAD, QF)
        hq, fr = _layer(hq, yq, m4, deg8, fr, conv, rep_mat)
        signals.append(fr[:N_NODES])

    out = _out_mlp(hq, params)[:N_NODES]
    return out, jnp.stack(signals)


# speedup vs baseline: 6.4339x; 1.5755x over previous
import functools

import jax
import jax.numpy as jnp
from jax import lax
from jax.experimental import pallas as pl
from jax.experimental.pallas import tpu as pltpu
from jax.experimental.pallas import tpu_sc as plsc

N_NODES = 10000
N_EDGES = 160000
IN_CH = 256
HID = 512
OUT_CH = 256
CHUNK = 64
NUM_LAYERS = 8
REP = HID // CHUNK

NPAD = 10240
RB = 256
NBLK = NPAD // RB
NQ = 4
QF = HID // NQ
QPC = NQ // 2

NC = 2
NS = 16
EK = 128
NCH = 80
HCH = NCH // 2
EPT = NCH * EK
NE_PAD = NS * EPT
ACC_ROWS = 10112
ACC_PT = ACC_ROWS // NS
TRASH = 10016
DW = 128


def _ln_blk(h, g, b, eps=1e-5):
    mu = jnp.mean(h, axis=-1, keepdims=True)
    d = h - mu
    var = jnp.mean(d * d, axis=-1, keepdims=True)
    return d / jnp.sqrt(var + eps) * g + b


def _dot(a, b):
    return jnp.dot(a, b, preferred_element_type=jnp.float32)


def _full(shape):
    nd = len(shape)
    return pl.BlockSpec(shape, lambda i: (0,) * nd)


def _planes(ref):
    return jnp.concatenate([ref[q] for q in range(NQ)], axis=-1)


def _store_planes(ref, val):
    for q in range(NQ):
        ref[q] = val[:, q * QF:(q + 1) * QF]


_PLANE_SPEC = pl.BlockSpec((NQ, RB, QF), lambda i: (0, i, 0))


def _stem_body(x_ref, w1, b1, g1, gb1, w2, b2, g2, gb2, o_ref):
    h = x_ref[...]
    h = jax.nn.gelu(_dot(h, w1[...]) + b1[...])
    h = _ln_blk(h, g1[...], gb1[...])
    h = jax.nn.gelu(_dot(h, w2[...]) + b2[...])
    h = _ln_blk(h, g2[...], gb2[...])
    _store_planes(o_ref, h + h)


def _stem(x, p):
    row = lambda a: a.reshape(1, -1)
    return pl.pallas_call(
        _stem_body,
        grid=(NBLK,),
        in_specs=[
            pl.BlockSpec((RB, IN_CH), lambda i: (i, 0)),
            _full((IN_CH, HID)), _full((1, HID)), _full((1, HID)), _full((1, HID)),
            _full((HID, HID)), _full((1, HID)), _full((1, HID)), _full((1, HID)),
        ],
        out_specs=_PLANE_SPEC,
        out_shape=jax.ShapeDtypeStruct((NQ, NPAD, QF), jnp.float32),
    )(x,
      p["lin_in"][0]["W"], row(p["lin_in"][0]["b"]),
      row(p["norm_in"][0]["g"]), row(p["norm_in"][0]["b"]),
      p["lin_in"][1]["W"], row(p["lin_in"][1]["b"]),
      row(p["norm_in"][1]["g"]), row(p["norm_in"][1]["b"]))


def _out_body(x_ref, w1, b1, w2, b2, o_ref):
    h = jax.nn.gelu(_dot(_planes(x_ref), w1[...]) + b1[...])
    o_ref[...] = _dot(h, w2[...]) + b2[...]


def _out_mlp(xq, p):
    row = lambda a: a.reshape(1, -1)
    return pl.pallas_call(
        _out_body,
        grid=(NBLK,),
        in_specs=[
            _PLANE_SPEC,
            _full((HID, HID)), _full((1, HID)),
            _full((HID, OUT_CH)), _full((1, OUT_CH)),
        ],
        out_specs=pl.BlockSpec((RB, OUT_CH), lambda i: (i, 0)),
        out_shape=jax.ShapeDtypeStruct((NPAD, OUT_CH), jnp.float32),
    )(xq, p["lin_out"][0]["W"], row(p["lin_out"][0]["b"]),
      p["lin_out"][1]["W"], row(p["lin_out"][1]["b"]))


def _layer_body(x_ref, y_ref, m4_ref, degA_ref, degB_ref, fr_ref,
                wi1, bi1, wi2, bi2, wf1, bf1, wf2, bf2,
                wo1, bo1, wo2, bo2, g, b, rep_ref,
                xo_ref, fro_ref):
    x = _planes(x_ref)
    msum = _planes(m4_ref)
    deg = degA_ref[...][:, 0:1] + degB_ref[...][:, 0:1]
    m = msum * (1.0 / jnp.clip(deg, 1.0, None))
    xm = jnp.concatenate([x, m], axis=-1)

    h = jax.nn.leaky_relu(_dot(xm, wi1[...]) + bi1[...], 0.01)
    in_sig = jax.nn.sigmoid(_dot(h, wi2[...]) + bi2[...])
    h = jax.nn.leaky_relu(_dot(xm, wf1[...]) + bf1[...], 0.01)
    fr_sig = jax.nn.sigmoid(_dot(h, wf2[...]) + bf2[...])

    fr_old = fr_ref[...]
    fr_new = fr_old + (1.0 - fr_old) * fr_sig
    rep = rep_ref[...]
    fr512 = _dot(fr_new, rep)
    ing512 = _dot(in_sig, rep)

    h = jax.nn.leaky_relu(_dot(m, wo1[...]) + bo1[...], 0.01)
    op = _dot(h, wo2[...]) + bo2[...]

    out = x * fr512 + op * (1.0 - fr512) * ing512 + _planes(y_ref)
    _store_planes(xo_ref, _ln_blk(out, g[...], b[...]))
    fro_ref[...] = fr_new


def _layer(xq, yq, m4, degA, degB, fr, conv, rep_mat):
    row = lambda a: a.reshape(1, -1)
    return pl.pallas_call(
        _layer_body,
        grid=(NBLK,),
        in_specs=[
            _PLANE_SPEC,
            _PLANE_SPEC,
            _PLANE_SPEC,
            pl.BlockSpec((RB, DW), lambda i: (i, 0)),
            pl.BlockSpec((RB, DW), lambda i: (i, 0)),
            pl.BlockSpec((RB, CHUNK), lambda i: (i, 0)),
            _full((2 * HID, CHUNK)), _full((1, CHUNK)),
            _full((CHUNK, CHUNK)), _full((1, CHUNK)),
            _full((2 * HID, CHUNK)), _full((1, CHUNK)),
            _full((CHUNK, CHUNK)), _full((1, CHUNK)),
            _full((HID, HID)), _full((1, HID)),
            _full((HID, HID)), _full((1, HID)),
            _full((1, HID)), _full((1, HID)),
            _full((CHUNK, HID)),
        ],
        out_specs=[
            _PLANE_SPEC,
            pl.BlockSpec((RB, CHUNK), lambda i: (i, 0)),
        ],
        out_shape=[
            jax.ShapeDtypeStruct((NQ, NPAD, QF), jnp.float32),
            jax.ShapeDtypeStruct((NPAD, CHUNK), jnp.float32),
        ],
    )(xq, yq, m4, degA, degB, fr,
      conv["in_net"][0]["W"], row(conv["in_net"][0]["b"]),
      conv["in_net"][1]["W"], row(conv["in_net"][1]["b"]),
      conv["fr_net"][0]["W"], row(conv["fr_net"][0]["b"]),
      conv["fr_net"][1]["W"], row(conv["fr_net"][1]["b"]),
      conv["op_net"][0]["W"], row(conv["op_net"][0]["b"]),
      conv["op_net"][1]["W"], row(conv["op_net"][1]["b"]),
      row(conv["tm_norm"]["g"]), row(conv["tm_norm"]["b"]),
      rep_mat)


def _deg_kernel(dst3):
    mesh = plsc.VectorSubcoreMesh(core_axis_name="c", subcore_axis_name="s")
    half = NCH // NC

    @functools.partial(
        pl.kernel,
        out_type=[jax.ShapeDtypeStruct((NPAD, DW), jnp.float32),
                  jax.ShapeDtypeStruct((NPAD, DW), jnp.float32)],
        mesh=mesh,
        scratch_types=[
            pltpu.VMEM((half, EK), jnp.int32),
            pltpu.VMEM((EK, DW), jnp.float32),
            pltpu.VMEM_SHARED((ACC_ROWS, DW), jnp.float32),
        ],
    )
    def run(dst_hbm, zeros_hbm, ones_hbm, outA, outB, dst_sl, ones_v, acc):
        c = lax.axis_index("c")
        s = lax.axis_index("s")
        pltpu.sync_copy(zeros_hbm, acc.at[pl.ds(s * ACC_PT, ACC_PT)])
        pltpu.sync_copy(dst_hbm.at[s].at[pl.ds(c * half, half)], dst_sl)
        pltpu.sync_copy(ones_hbm, ones_v)
        plsc.subcore_barrier()

        @pl.loop(0, half)
        def _(ch):
            pltpu.sync_copy(ones_v, acc.at[dst_sl.at[ch]], add=True)

        plsc.subcore_barrier()

        @pl.when(c == 0)
        def _():
            pltpu.sync_copy(acc.at[pl.ds(s * ACC_PT, ACC_PT)],
                            outA.at[pl.ds(s * ACC_PT, ACC_PT)])

        @pl.when(c == 1)
        def _():
            pltpu.sync_copy(acc.at[pl.ds(s * ACC_PT, ACC_PT)],
                            outB.at[pl.ds(s * ACC_PT, ACC_PT)])

    zeros = jnp.zeros((ACC_PT, DW), jnp.float32)
    ones = jnp.ones((EK, DW), jnp.float32)
    return run(dst3, zeros, ones)


def _segsum(xflat, idx4, dst3):
    mesh = plsc.VectorSubcoreMesh(core_axis_name="c", subcore_axis_name="s")

    @functools.partial(
        pl.kernel,
        out_type=jax.ShapeDtypeStruct((NQ * NPAD, QF), jnp.float32),
        mesh=mesh,
        scratch_types=[
            pltpu.VMEM((NCH * EK,), jnp.int32),
            pltpu.VMEM((HCH, EK), jnp.int32),
            pltpu.VMEM((EK, QF), jnp.float32),
            pltpu.VMEM((EK, QF), jnp.float32),
            pltpu.VMEM_SHARED((ACC_ROWS, QF), jnp.float32),
            pltpu.SemaphoreType.DMA,
            pltpu.SemaphoreType.DMA,
        ],
    )
    def run(x_hbm, idx_hbm, dst_hbm, zeros_hbm, out_hbm,
            idx_sl, dst_sl, rows_a, rows_b, acc, sem_a, sem_b):
        c = lax.axis_index("c")
        s = lax.axis_index("s")

        def gref(ch):
            return x_hbm.at[idx_sl.at[pl.ds(ch * EK, EK)]]

        for qi in range(QPC):
            q = c * QPC + qi
            pltpu.sync_copy(zeros_hbm, acc.at[pl.ds(s * ACC_PT, ACC_PT)])
            pltpu.sync_copy(idx_hbm.at[q * NS + s], idx_sl)
            plsc.subcore_barrier()

            for half in range(2):
                base = half * HCH
                pltpu.sync_copy(dst_hbm.at[s].at[pl.ds(base, HCH)], dst_sl)
                pltpu.async_copy(gref(base), rows_a, sem_a)

                @pl.loop(0, HCH // 2)
                def _(g):
                    l0 = 2 * g
                    ch0 = base + l0
                    pltpu.async_copy(gref(ch0 + 1), rows_b, sem_b)
                    pltpu.make_async_copy(gref(ch0), rows_a, sem_a).wait()
                    pltpu.sync_copy(rows_a, acc.at[dst_sl.at[l0]], add=True)

                    @pl.when(l0 + 2 < HCH)
                    def _():
                        pltpu.async_copy(gref(ch0 + 2), rows_a, sem_a)

                    pltpu.make_async_copy(gref(ch0 + 1), rows_b, sem_b).wait()
                    pltpu.sync_copy(rows_b, acc.at[dst_sl.at[l0 + 1]],
                                    add=True)

            plsc.subcore_barrier()
            pltpu.sync_copy(acc.at[pl.ds(s * ACC_PT, ACC_PT)],
                            out_hbm.at[pl.ds(q * NPAD + s * ACC_PT, ACC_PT)])

    zeros = jnp.zeros((ACC_PT, QF), jnp.float32)
    return run(xflat, idx4, dst3, zeros)


def kernel(x, edge_index, params):
    src = edge_index[0]
    dst = edge_index[1]
    npad_e = NE_PAD - N_EDGES
    fill = jnp.arange(npad_e, dtype=jnp.int32) % 64
    src_p = jnp.concatenate([src, fill])
    dst_p = jnp.concatenate([dst, TRASH + fill])
    idx4 = (jnp.arange(NQ, dtype=jnp.int32)[:, None] * NPAD + src_p[None, :]
            ).reshape(NQ * NS, NCH * EK)
    dst3 = dst_p.reshape(NS, NCH, EK)

    degA, degB = _deg_kernel(dst3)

    xp = jnp.pad(x, ((0, NPAD - N_NODES), (0, 0)))
    hq = _stem(xp, params)
    yq = hq

    rep_mat = jnp.repeat(jnp.eye(CHUNK, dtype=jnp.float32), REP, axis=1)
    fr = jnp.zeros((NPAD, CHUNK), jnp.float32)
    signals = []
    for conv in params["convs"]:
        msum = _segsum(hq.reshape(NQ * NPAD, QF), idx4, dst3)
        m4 = msum.reshape(NQ, NPAD, QF)
        hq, fr = _layer(hq, yq, m4, degA, degB, fr, conv, rep_mat)
        signals.append(fr[:N_NODES])

    out = _out_mlp(hq, params)[:N_NODES]
    return out, jnp.stack(signals)


# speedup vs baseline: 6.5598x; 1.0196x over previous
import functools

import jax
import jax.numpy as jnp
from jax import lax
from jax.experimental import pallas as pl
from jax.experimental.pallas import tpu as pltpu
from jax.experimental.pallas import tpu_sc as plsc

N_NODES = 10000
N_EDGES = 160000
IN_CH = 256
HID = 512
OUT_CH = 256
CHUNK = 64
NUM_LAYERS = 8
REP = HID // CHUNK

NPAD = 10240
RB = 256
NBLK = NPAD // RB
NQ = 4
QF = HID // NQ
QPC = NQ // 2

NC = 2
NS = 16
EK = 128
NCH = 80
HCH = NCH // 2
EPT = NCH * EK
NE_PAD = NS * EPT
ACC_ROWS = 10112
ACC_PT = ACC_ROWS // NS
TRASH = 10016
DW = 128


def _ln_blk(h, g, b, eps=1e-5):
    mu = jnp.mean(h, axis=-1, keepdims=True)
    d = h - mu
    var = jnp.mean(d * d, axis=-1, keepdims=True)
    return d / jnp.sqrt(var + eps) * g + b


def _dot(a, b):
    return jnp.dot(a, b, preferred_element_type=jnp.float32)


def _bdot(a, b_bf16):
    return jax.lax.dot_general(a.astype(jnp.bfloat16), b_bf16,
                               (((1,), (0,)), ((), ())),
                               preferred_element_type=jnp.float32)


def _full(shape):
    nd = len(shape)
    return pl.BlockSpec(shape, lambda i: (0,) * nd)


def _planes(ref):
    return jnp.concatenate([ref[q] for q in range(NQ)], axis=-1)


def _store_planes(ref, val):
    for q in range(NQ):
        ref[q] = val[:, q * QF:(q + 1) * QF]


_PLANE_SPEC = pl.BlockSpec((NQ, RB, QF), lambda i: (0, i, 0))


def _stem_body(x_ref, w1, b1, g1, gb1, w2, b2, g2, gb2, o_ref):
    h = x_ref[...]
    h = jax.nn.gelu(_dot(h, w1[...]) + b1[...])
    h = _ln_blk(h, g1[...], gb1[...])
    h = jax.nn.gelu(_dot(h, w2[...]) + b2[...])
    h = _ln_blk(h, g2[...], gb2[...])
    _store_planes(o_ref, h + h)


def _stem(x, p):
    row = lambda a: a.reshape(1, -1)
    return pl.pallas_call(
        _stem_body,
        grid=(NBLK,),
        in_specs=[
            pl.BlockSpec((RB, IN_CH), lambda i: (i, 0)),
            _full((IN_CH, HID)), _full((1, HID)), _full((1, HID)), _full((1, HID)),
            _full((HID, HID)), _full((1, HID)), _full((1, HID)), _full((1, HID)),
        ],
        out_specs=_PLANE_SPEC,
        out_shape=jax.ShapeDtypeStruct((NQ, NPAD, QF), jnp.float32),
    )(x,
      p["lin_in"][0]["W"], row(p["lin_in"][0]["b"]),
      row(p["norm_in"][0]["g"]), row(p["norm_in"][0]["b"]),
      p["lin_in"][1]["W"], row(p["lin_in"][1]["b"]),
      row(p["norm_in"][1]["g"]), row(p["norm_in"][1]["b"]))


def _out_body(x_ref, w1, b1, w2, b2, o_ref):
    h = jax.nn.gelu(_dot(_planes(x_ref), w1[...]) + b1[...])
    o_ref[...] = _dot(h, w2[...]) + b2[...]


def _out_mlp(xq, p):
    row = lambda a: a.reshape(1, -1)
    return pl.pallas_call(
        _out_body,
        grid=(NBLK,),
        in_specs=[
            _PLANE_SPEC,
            _full((HID, HID)), _full((1, HID)),
            _full((HID, OUT_CH)), _full((1, OUT_CH)),
        ],
        out_specs=pl.BlockSpec((RB, OUT_CH), lambda i: (i, 0)),
        out_shape=jax.ShapeDtypeStruct((NPAD, OUT_CH), jnp.float32),
    )(xq, p["lin_out"][0]["W"], row(p["lin_out"][0]["b"]),
      p["lin_out"][1]["W"], row(p["lin_out"][1]["b"]))


def _layer_body(x_ref, y_ref, m4_ref, degA_ref, degB_ref, fr_ref,
                wi1, bi1, wi2, bi2, wf1, bf1, wf2, bf2,
                wo1, bo1, wo2, bo2, g, b, rep_ref,
                xo_ref, fro_ref):
    x = _planes(x_ref)
    msum = _planes(m4_ref)
    deg = degA_ref[...][:, 0:1] + degB_ref[...][:, 0:1]
    m = msum * (1.0 / jnp.clip(deg, 1.0, None))
    xm = jnp.concatenate([x, m], axis=-1)

    h = jax.nn.leaky_relu(_bdot(xm, wi1[...]) + bi1[...], 0.01)
    in_sig = jax.nn.sigmoid(_bdot(h, wi2[...]) + bi2[...])
    h = jax.nn.leaky_relu(_bdot(xm, wf1[...]) + bf1[...], 0.01)
    fr_sig = jax.nn.sigmoid(_bdot(h, wf2[...]) + bf2[...])

    fr_old = fr_ref[...]
    fr_new = fr_old + (1.0 - fr_old) * fr_sig
    rep = rep_ref[...]
    fr512 = _dot(fr_new, rep)
    ing512 = _dot(in_sig, rep)

    h = jax.nn.leaky_relu(_bdot(m, wo1[...]) + bo1[...], 0.01)
    op = _bdot(h, wo2[...]) + bo2[...]

    out = x * fr512 + op * (1.0 - fr512) * ing512 + _planes(y_ref)
    _store_planes(xo_ref, _ln_blk(out, g[...], b[...]))
    fro_ref[...] = fr_new


def _layer(xq, yq, m4, degA, degB, fr, conv, rep_mat):
    row = lambda a: a.reshape(1, -1)
    bf = lambda a: a.astype(jnp.bfloat16)
    return pl.pallas_call(
        _layer_body,
        grid=(NBLK,),
        in_specs=[
            _PLANE_SPEC,
            _PLANE_SPEC,
            _PLANE_SPEC,
            pl.BlockSpec((RB, DW), lambda i: (i, 0)),
            pl.BlockSpec((RB, DW), lambda i: (i, 0)),
            pl.BlockSpec((RB, CHUNK), lambda i: (i, 0)),
            _full((2 * HID, CHUNK)), _full((1, CHUNK)),
            _full((CHUNK, CHUNK)), _full((1, CHUNK)),
            _full((2 * HID, CHUNK)), _full((1, CHUNK)),
            _full((CHUNK, CHUNK)), _full((1, CHUNK)),
            _full((HID, HID)), _full((1, HID)),
            _full((HID, HID)), _full((1, HID)),
            _full((1, HID)), _full((1, HID)),
            _full((CHUNK, HID)),
        ],
        out_specs=[
            _PLANE_SPEC,
            pl.BlockSpec((RB, CHUNK), lambda i: (i, 0)),
        ],
        out_shape=[
            jax.ShapeDtypeStruct((NQ, NPAD, QF), jnp.float32),
            jax.ShapeDtypeStruct((NPAD, CHUNK), jnp.float32),
        ],
    )(xq, yq, m4, degA, degB, fr,
      bf(conv["in_net"][0]["W"]), row(conv["in_net"][0]["b"]),
      bf(conv["in_net"][1]["W"]), row(conv["in_net"][1]["b"]),
      bf(conv["fr_net"][0]["W"]), row(conv["fr_net"][0]["b"]),
      bf(conv["fr_net"][1]["W"]), row(conv["fr_net"][1]["b"]),
      bf(conv["op_net"][0]["W"]), row(conv["op_net"][0]["b"]),
      bf(conv["op_net"][1]["W"]), row(conv["op_net"][1]["b"]),
      row(conv["tm_norm"]["g"]), row(conv["tm_norm"]["b"]),
      rep_mat)


def _deg_kernel(dst3):
    mesh = plsc.VectorSubcoreMesh(core_axis_name="c", subcore_axis_name="s")
    half = NCH // NC

    @functools.partial(
        pl.kernel,
        out_type=[jax.ShapeDtypeStruct((NPAD, DW), jnp.float32),
                  jax.ShapeDtypeStruct((NPAD, DW), jnp.float32)],
        mesh=mesh,
        scratch_types=[
            pltpu.VMEM((half, EK), jnp.int32),
            pltpu.VMEM((EK, DW), jnp.float32),
            pltpu.VMEM_SHARED((ACC_ROWS, DW), jnp.float32),
        ],
    )
    def run(dst_hbm, zeros_hbm, ones_hbm, outA, outB, dst_sl, ones_v, acc):
        c = lax.axis_index("c")
        s = lax.axis_index("s")
        pltpu.sync_copy(zeros_hbm, acc.at[pl.ds(s * ACC_PT, ACC_PT)])
        pltpu.sync_copy(dst_hbm.at[s].at[pl.ds(c * half, half)], dst_sl)
        pltpu.sync_copy(ones_hbm, ones_v)
        plsc.subcore_barrier()

        @pl.loop(0, half)
        def _(ch):
            pltpu.sync_copy(ones_v, acc.at[dst_sl.at[ch]], add=True)

        plsc.subcore_barrier()

        @pl.when(c == 0)
        def _():
            pltpu.sync_copy(acc.at[pl.ds(s * ACC_PT, ACC_PT)],
                            outA.at[pl.ds(s * ACC_PT, ACC_PT)])

        @pl.when(c == 1)
        def _():
            pltpu.sync_copy(acc.at[pl.ds(s * ACC_PT, ACC_PT)],
                            outB.at[pl.ds(s * ACC_PT, ACC_PT)])

    zeros = jnp.zeros((ACC_PT, DW), jnp.float32)
    ones = jnp.ones((EK, DW), jnp.float32)
    return run(dst3, zeros, ones)


def _segsum(xflat, idx4, dst3):
    mesh = plsc.VectorSubcoreMesh(core_axis_name="c", subcore_axis_name="s")

    @functools.partial(
        pl.kernel,
        out_type=jax.ShapeDtypeStruct((NQ * NPAD, QF), jnp.float32),
        mesh=mesh,
        scratch_types=[
            pltpu.VMEM((NCH * EK,), jnp.int32),
            pltpu.VMEM((HCH, EK), jnp.int32),
            pltpu.VMEM((EK, QF), jnp.float32),
            pltpu.VMEM((EK, QF), jnp.float32),
            pltpu.VMEM_SHARED((ACC_ROWS, QF), jnp.float32),
            pltpu.SemaphoreType.DMA,
            pltpu.SemaphoreType.DMA,
        ],
    )
    def run(x_hbm, idx_hbm, dst_hbm, zeros_hbm, out_hbm,
            idx_sl, dst_sl, rows_a, rows_b, acc, sem_a, sem_b):
        c = lax.axis_index("c")
        s = lax.axis_index("s")

        def gref(ch):
            return x_hbm.at[idx_sl.at[pl.ds(ch * EK, EK)]]

        for qi in range(QPC):
            q = c * QPC + qi
            pltpu.sync_copy(zeros_hbm, acc.at[pl.ds(s * ACC_PT, ACC_PT)])
            pltpu.sync_copy(idx_hbm.at[q * NS + s], idx_sl)
            plsc.subcore_barrier()

            for half in range(2):
                base = half * HCH
                pltpu.sync_copy(dst_hbm.at[s].at[pl.ds(base, HCH)], dst_sl)
                pltpu.async_copy(gref(base), rows_a, sem_a)

                @pl.loop(0, HCH // 2)
                def _(g):
                    l0 = 2 * g
                    ch0 = base + l0
                    pltpu.async_copy(gref(ch0 + 1), rows_b, sem_b)
                    pltpu.make_async_copy(gref(ch0), rows_a, sem_a).wait()
                    pltpu.sync_copy(rows_a, acc.at[dst_sl.at[l0]], add=True)

                    @pl.when(l0 + 2 < HCH)
                    def _():
                        pltpu.async_copy(gref(ch0 + 2), rows_a, sem_a)

                    pltpu.make_async_copy(gref(ch0 + 1), rows_b, sem_b).wait()
                    pltpu.sync_copy(rows_b, acc.at[dst_sl.at[l0 + 1]],
                                    add=True)

            plsc.subcore_barrier()
            pltpu.sync_copy(acc.at[pl.ds(s * ACC_PT, ACC_PT)],
                            out_hbm.at[pl.ds(q * NPAD + s * ACC_PT, ACC_PT)])

    zeros = jnp.zeros((ACC_PT, QF), jnp.float32)
    return run(xflat, idx4, dst3, zeros)


def kernel(x, edge_index, params):
    src = edge_index[0]
    dst = edge_index[1]
    npad_e = NE_PAD - N_EDGES
    fill = jnp.arange(npad_e, dtype=jnp.int32) % 64
    src_p = jnp.concatenate([src, fill])
    dst_p = jnp.concatenate([dst, TRASH + fill])
    idx4 = (jnp.arange(NQ, dtype=jnp.int32)[:, None] * NPAD + src_p[None, :]
            ).reshape(NQ * NS, NCH * EK)
    dst3 = dst_p.reshape(NS, NCH, EK)

    degA, degB = _deg_kernel(dst3)

    xp = jnp.pad(x, ((0, NPAD - N_NODES), (0, 0)))
    hq = _stem(xp, params)
    yq = hq

    rep_mat = jnp.repeat(jnp.eye(CHUNK, dtype=jnp.float32), REP, axis=1)
    fr = jnp.zeros((NPAD, CHUNK), jnp.float32)
    signals = []
    for conv in params["convs"]:
        msum = _segsum(hq.reshape(NQ * NPAD, QF), idx4, dst3)
        m4 = msum.reshape(NQ, NPAD, QF)
        hq, fr = _layer(hq, yq, m4, degA, degB, fr, conv, rep_mat)
        signals.append(fr[:N_NODES])

    out = _out_mlp(hq, params)[:N_NODES]
    return out, jnp.stack(signals)


# speedup vs baseline: 6.8792x; 1.0487x over previous
import functools

import jax
import jax.numpy as jnp
from jax import lax
from jax.experimental import pallas as pl
from jax.experimental.pallas import tpu as pltpu
from jax.experimental.pallas import tpu_sc as plsc

N_NODES = 10000
N_EDGES = 160000
IN_CH = 256
HID = 512
OUT_CH = 256
CHUNK = 64
NUM_LAYERS = 8
REP = HID // CHUNK

NPAD = 10240
RB = 512
NBLK = NPAD // RB
NQ = 4
QF = HID // NQ
QPC = NQ // 2

NC = 2
NS = 16
EK = 128
NCH = 80
HCH = NCH // 2
EPT = NCH * EK
NE_PAD = NS * EPT
ACC_ROWS = 10112
ACC_PT = ACC_ROWS // NS
TRASH = 10016
DW = 128


def _ln_blk(h, g, b, eps=1e-5):
    mu = jnp.mean(h, axis=-1, keepdims=True)
    d = h - mu
    var = jnp.mean(d * d, axis=-1, keepdims=True)
    return d / jnp.sqrt(var + eps) * g + b


def _dot(a, b):
    return jnp.dot(a, b, preferred_element_type=jnp.float32)


def _bdot(a, b_bf16):
    return jax.lax.dot_general(a.astype(jnp.bfloat16), b_bf16,
                               (((1,), (0,)), ((), ())),
                               preferred_element_type=jnp.float32)


def _full(shape):
    nd = len(shape)
    return pl.BlockSpec(shape, lambda i: (0,) * nd)


def _planes(ref):
    return jnp.concatenate([ref[q] for q in range(NQ)], axis=-1)


def _store_planes(ref, val):
    for q in range(NQ):
        ref[q] = val[:, q * QF:(q + 1) * QF]


_PLANE_SPEC = pl.BlockSpec((NQ, RB, QF), lambda i: (0, i, 0))


def _stem_body(x_ref, w1, b1, g1, gb1, w2, b2, g2, gb2, o_ref):
    h = x_ref[...]
    h = jax.nn.gelu(_dot(h, w1[...]) + b1[...])
    h = _ln_blk(h, g1[...], gb1[...])
    h = jax.nn.gelu(_dot(h, w2[...]) + b2[...])
    h = _ln_blk(h, g2[...], gb2[...])
    _store_planes(o_ref, h + h)


def _stem(x, p):
    row = lambda a: a.reshape(1, -1)
    return pl.pallas_call(
        _stem_body,
        grid=(NBLK,),
        in_specs=[
            pl.BlockSpec((RB, IN_CH), lambda i: (i, 0)),
            _full((IN_CH, HID)), _full((1, HID)), _full((1, HID)), _full((1, HID)),
            _full((HID, HID)), _full((1, HID)), _full((1, HID)), _full((1, HID)),
        ],
        out_specs=_PLANE_SPEC,
        out_shape=jax.ShapeDtypeStruct((NQ, NPAD, QF), jnp.float32),
    )(x,
      p["lin_in"][0]["W"], row(p["lin_in"][0]["b"]),
      row(p["norm_in"][0]["g"]), row(p["norm_in"][0]["b"]),
      p["lin_in"][1]["W"], row(p["lin_in"][1]["b"]),
      row(p["norm_in"][1]["g"]), row(p["norm_in"][1]["b"]))


def _out_body(x_ref, w1, b1, w2, b2, o_ref):
    h = jax.nn.gelu(_dot(_planes(x_ref), w1[...]) + b1[...])
    o_ref[...] = _dot(h, w2[...]) + b2[...]


def _out_mlp(xq, p):
    row = lambda a: a.reshape(1, -1)
    return pl.pallas_call(
        _out_body,
        grid=(NBLK,),
        in_specs=[
            _PLANE_SPEC,
            _full((HID, HID)), _full((1, HID)),
            _full((HID, OUT_CH)), _full((1, OUT_CH)),
        ],
        out_specs=pl.BlockSpec((RB, OUT_CH), lambda i: (i, 0)),
        out_shape=jax.ShapeDtypeStruct((NPAD, OUT_CH), jnp.float32),
    )(xq, p["lin_out"][0]["W"], row(p["lin_out"][0]["b"]),
      p["lin_out"][1]["W"], row(p["lin_out"][1]["b"]))


def _layer_body(x_ref, y_ref, m4_ref, degA_ref, degB_ref, fr_ref,
                wi1, bi1, wi2, bi2, wf1, bf1, wf2, bf2,
                wo1, bo1, wo2, bo2, g, b, rep_ref,
                xo_ref, fro_ref):
    x = _planes(x_ref)
    msum = _planes(m4_ref)
    deg = degA_ref[...][:, 0:1] + degB_ref[...][:, 0:1]
    m = msum * (1.0 / jnp.clip(deg, 1.0, None))
    xm = jnp.concatenate([x, m], axis=-1)

    h = jax.nn.leaky_relu(_bdot(xm, wi1[...]) + bi1[...], 0.01)
    in_sig = jax.nn.sigmoid(_bdot(h, wi2[...]) + bi2[...])
    h = jax.nn.leaky_relu(_bdot(xm, wf1[...]) + bf1[...], 0.01)
    fr_sig = jax.nn.sigmoid(_bdot(h, wf2[...]) + bf2[...])

    fr_old = fr_ref[...]
    fr_new = fr_old + (1.0 - fr_old) * fr_sig
    rep = rep_ref[...]
    fr512 = _dot(fr_new, rep)
    ing512 = _dot(in_sig, rep)

    h = jax.nn.leaky_relu(_bdot(m, wo1[...]) + bo1[...], 0.01)
    op = _bdot(h, wo2[...]) + bo2[...]

    out = x * fr512 + op * (1.0 - fr512) * ing512 + _planes(y_ref)
    _store_planes(xo_ref, _ln_blk(out, g[...], b[...]))
    fro_ref[...] = fr_new


def _layer(xq, yq, m4, degA, degB, fr, conv, rep_mat):
    row = lambda a: a.reshape(1, -1)
    bf = lambda a: a.astype(jnp.bfloat16)
    return pl.pallas_call(
        _layer_body,
        grid=(NBLK,),
        in_specs=[
            _PLANE_SPEC,
            _PLANE_SPEC,
            _PLANE_SPEC,
            pl.BlockSpec((RB, DW), lambda i: (i, 0)),
            pl.BlockSpec((RB, DW), lambda i: (i, 0)),
            pl.BlockSpec((RB, CHUNK), lambda i: (i, 0)),
            _full((2 * HID, CHUNK)), _full((1, CHUNK)),
            _full((CHUNK, CHUNK)), _full((1, CHUNK)),
            _full((2 * HID, CHUNK)), _full((1, CHUNK)),
            _full((CHUNK, CHUNK)), _full((1, CHUNK)),
            _full((HID, HID)), _full((1, HID)),
            _full((HID, HID)), _full((1, HID)),
            _full((1, HID)), _full((1, HID)),
            _full((CHUNK, HID)),
        ],
        out_specs=[
            _PLANE_SPEC,
            pl.BlockSpec((RB, CHUNK), lambda i: (i, 0)),
        ],
        out_shape=[
            jax.ShapeDtypeStruct((NQ, NPAD, QF), jnp.float32),
            jax.ShapeDtypeStruct((NPAD, CHUNK), jnp.float32),
        ],
    )(xq, yq, m4, degA, degB, fr,
      bf(conv["in_net"][0]["W"]), row(conv["in_net"][0]["b"]),
      bf(conv["in_net"][1]["W"]), row(conv["in_net"][1]["b"]),
      bf(conv["fr_net"][0]["W"]), row(conv["fr_net"][0]["b"]),
      bf(conv["fr_net"][1]["W"]), row(conv["fr_net"][1]["b"]),
      bf(conv["op_net"][0]["W"]), row(conv["op_net"][0]["b"]),
      bf(conv["op_net"][1]["W"]), row(conv["op_net"][1]["b"]),
      row(conv["tm_norm"]["g"]), row(conv["tm_norm"]["b"]),
      rep_mat)


def _deg_kernel(dst3):
    mesh = plsc.VectorSubcoreMesh(core_axis_name="c", subcore_axis_name="s")
    half = NCH // NC

    @functools.partial(
        pl.kernel,
        out_type=[jax.ShapeDtypeStruct((NPAD, DW), jnp.float32),
                  jax.ShapeDtypeStruct((NPAD, DW), jnp.float32)],
        mesh=mesh,
        scratch_types=[
            pltpu.VMEM((half, EK), jnp.int32),
            pltpu.VMEM((EK, DW), jnp.float32),
            pltpu.VMEM_SHARED((ACC_ROWS, DW), jnp.float32),
        ],
    )
    def run(dst_hbm, zeros_hbm, ones_hbm, outA, outB, dst_sl, ones_v, acc):
        c = lax.axis_index("c")
        s = lax.axis_index("s")
        pltpu.sync_copy(zeros_hbm, acc.at[pl.ds(s * ACC_PT, ACC_PT)])
        pltpu.sync_copy(dst_hbm.at[s].at[pl.ds(c * half, half)], dst_sl)
        pltpu.sync_copy(ones_hbm, ones_v)
        plsc.subcore_barrier()

        @pl.loop(0, half)
        def _(ch):
            pltpu.sync_copy(ones_v, acc.at[dst_sl.at[ch]], add=True)

        plsc.subcore_barrier()

        @pl.when(c == 0)
        def _():
            pltpu.sync_copy(acc.at[pl.ds(s * ACC_PT, ACC_PT)],
                            outA.at[pl.ds(s * ACC_PT, ACC_PT)])

        @pl.when(c == 1)
        def _():
            pltpu.sync_copy(acc.at[pl.ds(s * ACC_PT, ACC_PT)],
                            outB.at[pl.ds(s * ACC_PT, ACC_PT)])

    zeros = jnp.zeros((ACC_PT, DW), jnp.float32)
    ones = jnp.ones((EK, DW), jnp.float32)
    return run(dst3, zeros, ones)


def _segsum(xflat, idx4, dst3):
    mesh = plsc.VectorSubcoreMesh(core_axis_name="c", subcore_axis_name="s")

    @functools.partial(
        pl.kernel,
        out_type=jax.ShapeDtypeStruct((NQ * NPAD, QF), jnp.float32),
        mesh=mesh,
        scratch_types=[
            pltpu.VMEM((NCH * EK,), jnp.int32),
            pltpu.VMEM((HCH, EK), jnp.int32),
            pltpu.VMEM((EK, QF), jnp.float32),
            pltpu.VMEM((EK, QF), jnp.float32),
            pltpu.VMEM_SHARED((ACC_ROWS, QF), jnp.float32),
            pltpu.SemaphoreType.DMA,
            pltpu.SemaphoreType.DMA,
        ],
    )
    def run(x_hbm, idx_hbm, dst_hbm, zeros_hbm, out_hbm,
            idx_sl, dst_sl, rows_a, rows_b, acc, sem_a, sem_b):
        c = lax.axis_index("c")
        s = lax.axis_index("s")

        def gref(ch):
            return x_hbm.at[idx_sl.at[pl.ds(ch * EK, EK)]]

        for qi in range(QPC):
            q = c * QPC + qi
            pltpu.sync_copy(zeros_hbm, acc.at[pl.ds(s * ACC_PT, ACC_PT)])
            pltpu.sync_copy(idx_hbm.at[q * NS + s], idx_sl)
            plsc.subcore_barrier()

            for half in range(2):
                base = half * HCH
                pltpu.sync_copy(dst_hbm.at[s].at[pl.ds(base, HCH)], dst_sl)
                pltpu.async_copy(gref(base), rows_a, sem_a)

                @pl.loop(0, HCH // 2)
                def _(g):
                    l0 = 2 * g
                    ch0 = base + l0
                    pltpu.async_copy(gref(ch0 + 1), rows_b, sem_b)
                    pltpu.make_async_copy(gref(ch0), rows_a, sem_a).wait()
                    pltpu.sync_copy(rows_a, acc.at[dst_sl.at[l0]], add=True)

                    @pl.when(l0 + 2 < HCH)
                    def _():
                        pltpu.async_copy(gref(ch0 + 2), rows_a, sem_a)

                    pltpu.make_async_copy(gref(ch0 + 1), rows_b, sem_b).wait()
                    pltpu.sync_copy(rows_b, acc.at[dst_sl.at[l0 + 1]],
                                    add=True)

            plsc.subcore_barrier()
            pltpu.sync_copy(acc.at[pl.ds(s * ACC_PT, ACC_PT)],
                            out_hbm.at[pl.ds(q * NPAD + s * ACC_PT, ACC_PT)])

    zeros = jnp.zeros((ACC_PT, QF), jnp.float32)
    return run(xflat, idx4, dst3, zeros)


def kernel(x, edge_index, params):
    src = edge_index[0]
    dst = edge_index[1]
    npad_e = NE_PAD - N_EDGES
    fill = jnp.arange(npad_e, dtype=jnp.int32) % 64
    src_p = jnp.concatenate([src, fill])
    dst_p = jnp.concatenate([dst, TRASH + fill])
    idx4 = (jnp.arange(NQ, dtype=jnp.int32)[:, None] * NPAD + src_p[None, :]
            ).reshape(NQ * NS, NCH * EK)
    dst3 = dst_p.reshape(NS, NCH, EK)

    degA, degB = _deg_kernel(dst3)

    xp = jnp.pad(x, ((0, NPAD - N_NODES), (0, 0)))
    hq = _stem(xp, params)
    yq = hq

    rep_mat = jnp.repeat(jnp.eye(CHUNK, dtype=jnp.float32), REP, axis=1)
    fr = jnp.zeros((NPAD, CHUNK), jnp.float32)
    signals = []
    for conv in params["convs"]:
        msum = _segsum(hq.reshape(NQ * NPAD, QF), idx4, dst3)
        m4 = msum.reshape(NQ, NPAD, QF)
        hq, fr = _layer(hq, yq, m4, degA, degB, fr, conv, rep_mat)
        signals.append(fr[:N_NODES])

    out = _out_mlp(hq, params)[:N_NODES]
    return out, jnp.stack(signals)


# speedup vs baseline: 6.9909x; 1.0162x over previous
import functools

import jax
import jax.numpy as jnp
from jax import lax
from jax.experimental import pallas as pl
from jax.experimental.pallas import tpu as pltpu
from jax.experimental.pallas import tpu_sc as plsc

N_NODES = 10000
N_EDGES = 160000
IN_CH = 256
HID = 512
OUT_CH = 256
CHUNK = 64
NUM_LAYERS = 8
REP = HID // CHUNK

NPAD = 10240
RB = 1024
NBLK = NPAD // RB
NQ = 4
QF = HID // NQ
QPC = NQ // 2

NC = 2
NS = 16
EK = 128
NCH = 80
HCH = NCH // 2
EPT = NCH * EK
NE_PAD = NS * EPT
ACC_ROWS = 10112
ACC_PT = ACC_ROWS // NS
TRASH = 10016
DW = 128


def _ln_blk(h, g, b, eps=1e-5):
    mu = jnp.mean(h, axis=-1, keepdims=True)
    d = h - mu
    var = jnp.mean(d * d, axis=-1, keepdims=True)
    return d / jnp.sqrt(var + eps) * g + b


def _dot(a, b):
    return jnp.dot(a, b, preferred_element_type=jnp.float32)


def _bdot(a, b_bf16):
    return jax.lax.dot_general(a.astype(jnp.bfloat16), b_bf16,
                               (((1,), (0,)), ((), ())),
                               preferred_element_type=jnp.float32)


def _full(shape):
    nd = len(shape)
    return pl.BlockSpec(shape, lambda i: (0,) * nd)


def _planes(ref):
    return jnp.concatenate([ref[q] for q in range(NQ)], axis=-1)


def _store_planes(ref, val):
    for q in range(NQ):
        ref[q] = val[:, q * QF:(q + 1) * QF]


_PLANE_SPEC = pl.BlockSpec((NQ, RB, QF), lambda i: (0, i, 0))


def _stem_body(x_ref, w1, b1, g1, gb1, w2, b2, g2, gb2, o_ref):
    h = x_ref[...]
    h = jax.nn.gelu(_dot(h, w1[...]) + b1[...])
    h = _ln_blk(h, g1[...], gb1[...])
    h = jax.nn.gelu(_dot(h, w2[...]) + b2[...])
    h = _ln_blk(h, g2[...], gb2[...])
    _store_planes(o_ref, h + h)


def _stem(x, p):
    row = lambda a: a.reshape(1, -1)
    return pl.pallas_call(
        _stem_body,
        grid=(NBLK,),
        in_specs=[
            pl.BlockSpec((RB, IN_CH), lambda i: (i, 0)),
            _full((IN_CH, HID)), _full((1, HID)), _full((1, HID)), _full((1, HID)),
            _full((HID, HID)), _full((1, HID)), _full((1, HID)), _full((1, HID)),
        ],
        out_specs=_PLANE_SPEC,
        out_shape=jax.ShapeDtypeStruct((NQ, NPAD, QF), jnp.float32),
    )(x,
      p["lin_in"][0]["W"], row(p["lin_in"][0]["b"]),
      row(p["norm_in"][0]["g"]), row(p["norm_in"][0]["b"]),
      p["lin_in"][1]["W"], row(p["lin_in"][1]["b"]),
      row(p["norm_in"][1]["g"]), row(p["norm_in"][1]["b"]))


def _out_body(x_ref, w1, b1, w2, b2, o_ref):
    h = jax.nn.gelu(_dot(_planes(x_ref), w1[...]) + b1[...])
    o_ref[...] = _dot(h, w2[...]) + b2[...]


def _out_mlp(xq, p):
    row = lambda a: a.reshape(1, -1)
    return pl.pallas_call(
        _out_body,
        grid=(NBLK,),
        in_specs=[
            _PLANE_SPEC,
            _full((HID, HID)), _full((1, HID)),
            _full((HID, OUT_CH)), _full((1, OUT_CH)),
        ],
        out_specs=pl.BlockSpec((RB, OUT_CH), lambda i: (i, 0)),
        out_shape=jax.ShapeDtypeStruct((NPAD, OUT_CH), jnp.float32),
    )(xq, p["lin_out"][0]["W"], row(p["lin_out"][0]["b"]),
      p["lin_out"][1]["W"], row(p["lin_out"][1]["b"]))


def _layer_body(x_ref, y_ref, m4_ref, degA_ref, degB_ref, fr_ref,
                wi1, bi1, wi2, bi2, wf1, bf1, wf2, bf2,
                wo1, bo1, wo2, bo2, g, b, rep_ref,
                xo_ref, fro_ref):
    x = _planes(x_ref)
    msum = _planes(m4_ref)
    deg = degA_ref[...][:, 0:1] + degB_ref[...][:, 0:1]
    m = msum * (1.0 / jnp.clip(deg, 1.0, None))
    xm = jnp.concatenate([x, m], axis=-1)

    h = jax.nn.leaky_relu(_bdot(xm, wi1[...]) + bi1[...], 0.01)
    in_sig = jax.nn.sigmoid(_bdot(h, wi2[...]) + bi2[...])
    h = jax.nn.leaky_relu(_bdot(xm, wf1[...]) + bf1[...], 0.01)
    fr_sig = jax.nn.sigmoid(_bdot(h, wf2[...]) + bf2[...])

    fr_old = fr_ref[...]
    fr_new = fr_old + (1.0 - fr_old) * fr_sig
    rep = rep_ref[...]
    fr512 = _dot(fr_new, rep)
    ing512 = _dot(in_sig, rep)

    h = jax.nn.leaky_relu(_bdot(m, wo1[...]) + bo1[...], 0.01)
    op = _bdot(h, wo2[...]) + bo2[...]

    out = x * fr512 + op * (1.0 - fr512) * ing512 + _planes(y_ref)
    _store_planes(xo_ref, _ln_blk(out, g[...], b[...]))
    fro_ref[...] = fr_new


def _layer(xq, yq, m4, degA, degB, fr, conv, rep_mat):
    row = lambda a: a.reshape(1, -1)
    bf = lambda a: a.astype(jnp.bfloat16)
    return pl.pallas_call(
        _layer_body,
        grid=(NBLK,),
        in_specs=[
            _PLANE_SPEC,
            _PLANE_SPEC,
            _PLANE_SPEC,
            pl.BlockSpec((RB, DW), lambda i: (i, 0)),
            pl.BlockSpec((RB, DW), lambda i: (i, 0)),
            pl.BlockSpec((RB, CHUNK), lambda i: (i, 0)),
            _full((2 * HID, CHUNK)), _full((1, CHUNK)),
            _full((CHUNK, CHUNK)), _full((1, CHUNK)),
            _full((2 * HID, CHUNK)), _full((1, CHUNK)),
            _full((CHUNK, CHUNK)), _full((1, CHUNK)),
            _full((HID, HID)), _full((1, HID)),
            _full((HID, HID)), _full((1, HID)),
            _full((1, HID)), _full((1, HID)),
            _full((CHUNK, HID)),
        ],
        out_specs=[
            _PLANE_SPEC,
            pl.BlockSpec((RB, CHUNK), lambda i: (i, 0)),
        ],
        out_shape=[
            jax.ShapeDtypeStruct((NQ, NPAD, QF), jnp.float32),
            jax.ShapeDtypeStruct((NPAD, CHUNK), jnp.float32),
        ],
    )(xq, yq, m4, degA, degB, fr,
      bf(conv["in_net"][0]["W"]), row(conv["in_net"][0]["b"]),
      bf(conv["in_net"][1]["W"]), row(conv["in_net"][1]["b"]),
      bf(conv["fr_net"][0]["W"]), row(conv["fr_net"][0]["b"]),
      bf(conv["fr_net"][1]["W"]), row(conv["fr_net"][1]["b"]),
      bf(conv["op_net"][0]["W"]), row(conv["op_net"][0]["b"]),
      bf(conv["op_net"][1]["W"]), row(conv["op_net"][1]["b"]),
      row(conv["tm_norm"]["g"]), row(conv["tm_norm"]["b"]),
      rep_mat)


def _deg_kernel(dst3):
    mesh = plsc.VectorSubcoreMesh(core_axis_name="c", subcore_axis_name="s")
    half = NCH // NC

    @functools.partial(
        pl.kernel,
        out_type=[jax.ShapeDtypeStruct((NPAD, DW), jnp.float32),
                  jax.ShapeDtypeStruct((NPAD, DW), jnp.float32)],
        mesh=mesh,
        scratch_types=[
            pltpu.VMEM((half, EK), jnp.int32),
            pltpu.VMEM((EK, DW), jnp.float32),
            pltpu.VMEM_SHARED((ACC_ROWS, DW), jnp.float32),
        ],
    )
    def run(dst_hbm, zeros_hbm, ones_hbm, outA, outB, dst_sl, ones_v, acc):
        c = lax.axis_index("c")
        s = lax.axis_index("s")
        pltpu.sync_copy(zeros_hbm, acc.at[pl.ds(s * ACC_PT, ACC_PT)])
        pltpu.sync_copy(dst_hbm.at[s].at[pl.ds(c * half, half)], dst_sl)
        pltpu.sync_copy(ones_hbm, ones_v)
        plsc.subcore_barrier()

        @pl.loop(0, half)
        def _(ch):
            pltpu.sync_copy(ones_v, acc.at[dst_sl.at[ch]], add=True)

        plsc.subcore_barrier()

        @pl.when(c == 0)
        def _():
            pltpu.sync_copy(acc.at[pl.ds(s * ACC_PT, ACC_PT)],
                            outA.at[pl.ds(s * ACC_PT, ACC_PT)])

        @pl.when(c == 1)
        def _():
            pltpu.sync_copy(acc.at[pl.ds(s * ACC_PT, ACC_PT)],
                            outB.at[pl.ds(s * ACC_PT, ACC_PT)])

    zeros = jnp.zeros((ACC_PT, DW), jnp.float32)
    ones = jnp.ones((EK, DW), jnp.float32)
    return run(dst3, zeros, ones)


def _segsum(xflat, idx4, dst3):
    mesh = plsc.VectorSubcoreMesh(core_axis_name="c", subcore_axis_name="s")

    @functools.partial(
        pl.kernel,
        out_type=jax.ShapeDtypeStruct((NQ * NPAD, QF), jnp.float32),
        mesh=mesh,
        scratch_types=[
            pltpu.VMEM((NCH * EK,), jnp.int32),
            pltpu.VMEM((HCH, EK), jnp.int32),
            pltpu.VMEM((EK, QF), jnp.float32),
            pltpu.VMEM((EK, QF), jnp.float32),
            pltpu.VMEM_SHARED((ACC_ROWS, QF), jnp.float32),
            pltpu.SemaphoreType.DMA,
            pltpu.SemaphoreType.DMA,
        ],
    )
    def run(x_hbm, idx_hbm, dst_hbm, zeros_hbm, out_hbm,
            idx_sl, dst_sl, rows_a, rows_b, acc, sem_a, sem_b):
        c = lax.axis_index("c")
        s = lax.axis_index("s")

        def gref(ch):
            return x_hbm.at[idx_sl.at[pl.ds(ch * EK, EK)]]

        for qi in range(QPC):
            q = c * QPC + qi
            pltpu.sync_copy(zeros_hbm, acc.at[pl.ds(s * ACC_PT, ACC_PT)])
            pltpu.sync_copy(idx_hbm.at[q * NS + s], idx_sl)
            plsc.subcore_barrier()

            for half in range(2):
                base = half * HCH
                pltpu.sync_copy(dst_hbm.at[s].at[pl.ds(base, HCH)], dst_sl)
                pltpu.async_copy(gref(base), rows_a, sem_a)

                @pl.loop(0, HCH // 2)
                def _(g):
                    l0 = 2 * g
                    ch0 = base + l0
                    pltpu.async_copy(gref(ch0 + 1), rows_b, sem_b)
                    pltpu.make_async_copy(gref(ch0), rows_a, sem_a).wait()
                    pltpu.sync_copy(rows_a, acc.at[dst_sl.at[l0]], add=True)

                    @pl.when(l0 + 2 < HCH)
                    def _():
                        pltpu.async_copy(gref(ch0 + 2), rows_a, sem_a)

                    pltpu.make_async_copy(gref(ch0 + 1), rows_b, sem_b).wait()
                    pltpu.sync_copy(rows_b, acc.at[dst_sl.at[l0 + 1]],
                                    add=True)

            plsc.subcore_barrier()
            pltpu.sync_copy(acc.at[pl.ds(s * ACC_PT, ACC_PT)],
                            out_hbm.at[pl.ds(q * NPAD + s * ACC_PT, ACC_PT)])

    zeros = jnp.zeros((ACC_PT, QF), jnp.float32)
    return run(xflat, idx4, dst3, zeros)


def kernel(x, edge_index, params):
    src = edge_index[0]
    dst = edge_index[1]
    npad_e = NE_PAD - N_EDGES
    fill = jnp.arange(npad_e, dtype=jnp.int32) % 64
    src_p = jnp.concatenate([src, fill])
    dst_p = jnp.concatenate([dst, TRASH + fill])
    idx4 = (jnp.arange(NQ, dtype=jnp.int32)[:, None] * NPAD + src_p[None, :]
            ).reshape(NQ * NS, NCH * EK)
    dst3 = dst_p.reshape(NS, NCH, EK)

    degA, degB = _deg_kernel(dst3)

    xp = jnp.pad(x, ((0, NPAD - N_NODES), (0, 0)))
    hq = _stem(xp, params)
    yq = hq

    rep_mat = jnp.repeat(jnp.eye(CHUNK, dtype=jnp.float32), REP, axis=1)
    fr = jnp.zeros((NPAD, CHUNK), jnp.float32)
    signals = []
    for conv in params["convs"]:
        msum = _segsum(hq.reshape(NQ * NPAD, QF), idx4, dst3)
        m4 = msum.reshape(NQ, NPAD, QF)
        hq, fr = _layer(hq, yq, m4, degA, degB, fr, conv, rep_mat)
        signals.append(fr[:N_NODES])

    out = _out_mlp(hq, params)[:N_NODES]
    return out, jnp.stack(signals)
